# baseline (speedup 1.0000x reference)
import functools

import jax
import jax.numpy as jnp
from jax import lax
from jax.experimental import pallas as pl
from jax.experimental.pallas import tpu as pltpu
from jax.experimental.pallas import tpu_sc as plsc

_L = 16
_NC, _NS = 2, 16
_NW = _NC * _NS
_N = 64 * 32768
_X1_PER_W = _N // _NW
_ROW = 32768
_CHUNK = 16384
_BIG = 2**31 - 1


def _scan_chunk(buf, chunk_base, best, besti):
    lanes = lax.broadcasted_iota(jnp.int32, (_L,), 0)

    def body(i, carry):
        b, bi = carry
        v = buf[pl.ds(i * _L, _L)]
        idx = chunk_base + i * _L + lanes
        m = v > b
        return jnp.where(m, v, b), jnp.where(m, idx, bi)

    return lax.fori_loop(0, _CHUNK // _L, body, (best, besti))


def _sc_body(x1_hbm, x2_hbm, pval_hbm, pidx_hbm, rowval_hbm, rowidx_hbm,
             buf, pvb, pib, rvb, rib):
    wid = lax.axis_index("s") * _NC + lax.axis_index("c")
    neg_inf = jnp.full((_L,), -jnp.inf, jnp.float32)
    zeros = jnp.zeros((_L,), jnp.int32)

    best, besti = neg_inf, zeros
    base1 = wid * _X1_PER_W
    for ch in range(_X1_PER_W // _CHUNK):
        cb = base1 + ch * _CHUNK
        pltpu.sync_copy(x1_hbm.at[pl.ds(cb, _CHUNK)], buf)
        best, besti = _scan_chunk(buf, cb, best, besti)
    pvb[...] = best
    pib[...] = besti
    pltpu.sync_copy(pvb, pval_hbm.at[wid])
    pltpu.sync_copy(pib, pidx_hbm.at[wid])

    for j in range(2):
        rbase = (2 * wid + j) * _ROW
        best, besti = neg_inf, zeros
        for ch in range(_ROW // _CHUNK):
            pltpu.sync_copy(x2_hbm.at[pl.ds(rbase + ch * _CHUNK, _CHUNK)], buf)
            best, besti = _scan_chunk(buf, ch * _CHUNK, best, besti)
        rvb[j] = best
        rib[j] = besti
    pltpu.sync_copy(rvb, rowval_hbm.at[pl.ds(2 * wid, 2)])
    pltpu.sync_copy(rib, rowidx_hbm.at[pl.ds(2 * wid, 2)])


_sc_partial = functools.partial(
    pl.kernel,
    out_type=[
        jax.ShapeDtypeStruct((_NW, _L), jnp.float32),
        jax.ShapeDtypeStruct((_NW, _L), jnp.int32),
        jax.ShapeDtypeStruct((64, _L), jnp.float32),
        jax.ShapeDtypeStruct((64, _L), jnp.int32),
    ],
    mesh=plsc.VectorSubcoreMesh(core_axis_name="c", subcore_axis_name="s"),
    scratch_types=[
        pltpu.VMEM((_CHUNK,), jnp.float32),
        pltpu.VMEM((_L,), jnp.float32),
        pltpu.VMEM((_L,), jnp.int32),
        pltpu.VMEM((2, _L), jnp.float32),
        pltpu.VMEM((2, _L), jnp.int32),
    ],
)(_sc_body)


def _merge_body(pv_ref, pi_ref, rv_ref, ri_ref, o_ref):
    mv = jnp.max(pv_ref[...])
    gidx = jnp.min(jnp.where(pv_ref[...] == mv, pi_ref[...], _BIG))
    rmax = jnp.max(rv_ref[...], axis=1, keepdims=True)
    rarg = jnp.min(jnp.where(rv_ref[...] == rmax, ri_ref[...], _BIG), axis=1)
    o_ref[...] = rarg + gidx


def kernel(x1, x2):
    pvals, pidx, rowval, rowidx = _sc_partial(x1.reshape(-1), x2.reshape(-1))
    return pl.pallas_call(
        _merge_body,
        out_shape=jax.ShapeDtypeStruct((64,), jnp.int32),
    )(pvals, pidx, rowval, rowidx)


# speedup vs baseline: 1.6244x; 1.6244x over previous
import functools

import jax
import jax.numpy as jnp
from jax import lax
from jax.experimental import pallas as pl
from jax.experimental.pallas import tpu as pltpu
from jax.experimental.pallas import tpu_sc as plsc

_L = 16
_NC, _NS = 2, 16
_NW = _NC * _NS
_N = 64 * 32768
_X1_PER_W = _N // _NW
_ROW = 32768
_CHUNK = 32768
_UNROLL = 8
_BIG = 2**31 - 1


def _scan_chunk(buf):
    def body(i, carry):
        b, s = carry
        for u in range(_UNROLL):
            step = i * _UNROLL + u
            v = buf[pl.ds(step * _L, _L)]
            m = v > b
            b = jnp.where(m, v, b)
            s = jnp.where(m, step, s)
        return b, s

    init = (jnp.full((_L,), -jnp.inf, jnp.float32), jnp.zeros((_L,), jnp.int32))
    return lax.fori_loop(0, _CHUNK // _L // _UNROLL, body, init)


def _sc_body(x1_hbm, x2_hbm, pval_hbm, pidx_hbm, rowval_hbm, rowidx_hbm,
             buf0, buf1, pvb, pib, rvb, rib, sem0, sem1):
    wid = lax.axis_index("s") * _NC + lax.axis_index("c")
    lanes = lax.broadcasted_iota(jnp.int32, (_L,), 0)
    bufs, sems = (buf0, buf1), (sem0, sem1)

    base1 = wid * _X1_PER_W
    descs = [
        (x1_hbm, base1, 0, 0),
        (x1_hbm, base1 + _CHUNK, 0, _CHUNK),
        (x2_hbm, (2 * wid) * _ROW, 1, 0),
        (x2_hbm, (2 * wid + 1) * _ROW, 2, 0),
    ]
    nk = len(descs)
    handles = [None] * nk

    def issue(k):
        ref, off = descs[k][0], descs[k][1]
        handles[k] = pltpu.async_copy(ref.at[pl.ds(off, _CHUNK)], bufs[k % 2],
                                      sems[k % 2])

    issue(0)
    issue(1)
    seg = {}
    for k in range(nk):
        handles[k].wait()
        bc, sc = _scan_chunk(bufs[k % 2])
        if k + 2 < nk:
            issue(k + 2)
        absi = descs[k][3] + (sc << 4) + lanes
        s = descs[k][2]
        if s in seg:
            b0, i0 = seg[s]
            m = bc > b0
            seg[s] = (jnp.where(m, bc, b0), jnp.where(m, absi, i0))
        else:
            seg[s] = (bc, absi)

    pvb[...] = seg[0][0]
    pib[...] = base1 + seg[0][1]
    for j in range(2):
        rvb[j] = seg[1 + j][0]
        rib[j] = seg[1 + j][1]
    pltpu.sync_copy(pvb, pval_hbm.at[wid])
    pltpu.sync_copy(pib, pidx_hbm.at[wid])
    pltpu.sync_copy(rvb, rowval_hbm.at[pl.ds(2 * wid, 2)])
    pltpu.sync_copy(rib, rowidx_hbm.at[pl.ds(2 * wid, 2)])


_sc_partial = functools.partial(
    pl.kernel,
    out_type=[
        jax.ShapeDtypeStruct((_NW, _L), jnp.float32),
        jax.ShapeDtypeStruct((_NW, _L), jnp.int32),
        jax.ShapeDtypeStruct((64, _L), jnp.float32),
        jax.ShapeDtypeStruct((64, _L), jnp.int32),
    ],
    mesh=plsc.VectorSubcoreMesh(core_axis_name="c", subcore_axis_name="s"),
    scratch_types=[
        pltpu.VMEM((_CHUNK,), jnp.float32),
        pltpu.VMEM((_CHUNK,), jnp.float32),
        pltpu.VMEM((_L,), jnp.float32),
        pltpu.VMEM((_L,), jnp.int32),
        pltpu.VMEM((2, _L), jnp.float32),
        pltpu.VMEM((2, _L), jnp.int32),
        pltpu.SemaphoreType.DMA,
        pltpu.SemaphoreType.DMA,
    ],
)(_sc_body)


def _merge_body(pv_ref, pi_ref, rv_ref, ri_ref, o_ref):
    mv = jnp.max(pv_ref[...])
    gidx = jnp.min(jnp.where(pv_ref[...] == mv, pi_ref[...], _BIG))
    rmax = jnp.max(rv_ref[...], axis=1, keepdims=True)
    rarg = jnp.min(jnp.where(rv_ref[...] == rmax, ri_ref[...], _BIG), axis=1)
    o_ref[...] = rarg + gidx


def kernel(x1, x2):
    pvals, pidx, rowval, rowidx = _sc_partial(x1.reshape(-1), x2.reshape(-1))
    return pl.pallas_call(
        _merge_body,
        out_shape=jax.ShapeDtypeStruct((64,), jnp.int32),
    )(pvals, pidx, rowval, rowidx)


# speedup vs baseline: 2.6635x; 1.6397x over previous
import functools

import jax
import jax.numpy as jnp
from jax import lax
from jax.experimental import pallas as pl
from jax.experimental.pallas import tpu as pltpu
from jax.experimental.pallas import tpu_sc as plsc

_L = 16
_NC, _NS = 2, 16
_NW = _NC * _NS
_N = 64 * 32768
_X1_PER_W = _N // _NW
_ROW = 32768
_CHUNK = 32768
_UNROLL = 8
_BIG = 2**31 - 1


def _scan_chunk(buf):
    def body(i, carry):
        b, s = carry
        for u in range(_UNROLL):
            step = i * _UNROLL + u
            v = buf[pl.ds(step * _L, _L)]
            m = v > b
            b = jnp.where(m, v, b)
            s = jnp.where(m, step, s)
        return b, s

    init = (jnp.full((_L,), -jnp.inf, jnp.float32), jnp.zeros((_L,), jnp.int32))
    return lax.fori_loop(0, _CHUNK // _L // _UNROLL, body, init)


def _sc_body(x1_hbm, x2_hbm, pval_hbm, pidx_hbm, rowval_hbm, rowidx_hbm,
             buf0, buf1, pvb, pib, rvb, rib, sem0, sem1):
    wid = lax.axis_index("s") * _NC + lax.axis_index("c")
    lanes = lax.broadcasted_iota(jnp.int32, (_L,), 0)
    bufs, sems = (buf0, buf1), (sem0, sem1)

    base1 = wid * _X1_PER_W
    descs = [
        (x1_hbm, 2 * wid, 0, 0),
        (x1_hbm, 2 * wid + 1, 0, _CHUNK),
        (x2_hbm, 2 * wid, 1, 0),
        (x2_hbm, 2 * wid + 1, 2, 0),
    ]
    nk = len(descs)
    handles = [None] * nk

    def issue(k):
        ref, row = descs[k][0], descs[k][1]
        handles[k] = pltpu.async_copy(ref.at[row], bufs[k % 2], sems[k % 2])

    issue(0)
    issue(1)
    seg = {}
    for k in range(nk):
        handles[k].wait()
        bc, sc = _scan_chunk(bufs[k % 2])
        if k + 2 < nk:
            issue(k + 2)
        absi = descs[k][3] + (sc << 4) + lanes
        s = descs[k][2]
        if s in seg:
            b0, i0 = seg[s]
            m = bc > b0
            seg[s] = (jnp.where(m, bc, b0), jnp.where(m, absi, i0))
        else:
            seg[s] = (bc, absi)

    pvb[...] = seg[0][0]
    pib[...] = base1 + seg[0][1]
    for j in range(2):
        rvb[j] = seg[1 + j][0]
        rib[j] = seg[1 + j][1]
    pltpu.sync_copy(pvb, pval_hbm.at[wid])
    pltpu.sync_copy(pib, pidx_hbm.at[wid])
    pltpu.sync_copy(rvb, rowval_hbm.at[pl.ds(2 * wid, 2)])
    pltpu.sync_copy(rib, rowidx_hbm.at[pl.ds(2 * wid, 2)])


_sc_partial = functools.partial(
    pl.kernel,
    out_type=[
        jax.ShapeDtypeStruct((_NW, _L), jnp.float32),
        jax.ShapeDtypeStruct((_NW, _L), jnp.int32),
        jax.ShapeDtypeStruct((64, _L), jnp.float32),
        jax.ShapeDtypeStruct((64, _L), jnp.int32),
    ],
    mesh=plsc.VectorSubcoreMesh(core_axis_name="c", subcore_axis_name="s"),
    scratch_types=[
        pltpu.VMEM((_CHUNK,), jnp.float32),
        pltpu.VMEM((_CHUNK,), jnp.float32),
        pltpu.VMEM((_L,), jnp.float32),
        pltpu.VMEM((_L,), jnp.int32),
        pltpu.VMEM((2, _L), jnp.float32),
        pltpu.VMEM((2, _L), jnp.int32),
        pltpu.SemaphoreType.DMA,
        pltpu.SemaphoreType.DMA,
    ],
)(_sc_body)


def _merge_body(pv_ref, pi_ref, rv_ref, ri_ref, o_ref):
    mv = jnp.max(pv_ref[...])
    gidx = jnp.min(jnp.where(pv_ref[...] == mv, pi_ref[...], _BIG))
    rmax = jnp.max(rv_ref[...], axis=1, keepdims=True)
    rarg = jnp.min(jnp.where(rv_ref[...] == rmax, ri_ref[...], _BIG), axis=1)
    o_ref[...] = rarg + gidx


def kernel(x1, x2):
    pvals, pidx, rowval, rowidx = _sc_partial(x1, x2)
    return pl.pallas_call(
        _merge_body,
        out_shape=jax.ShapeDtypeStruct((64,), jnp.int32),
    )(pvals, pidx, rowval, rowidx)


# speedup vs baseline: 2.9560x; 1.1098x over previous
import functools

import jax
import jax.numpy as jnp
from jax import lax
from jax.experimental import pallas as pl
from jax.experimental.pallas import tpu as pltpu
from jax.experimental.pallas import tpu_sc as plsc

_L = 16
_NC, _NS = 2, 16
_NW = _NC * _NS
_N = 64 * 32768
_X1_PER_W = _N // _NW
_ROW = 32768
_CHUNK = 32768
_UNROLL = 8
_BIG = 2**31 - 1


def _scan_chunk(buf):
    U = _UNROLL

    def body(i, carry):
        bs, ss = list(carry[0]), list(carry[1])
        for u in range(U):
            v = buf[pl.ds((i * U + u) * _L, _L)]
            m = v > bs[u]
            bs[u] = jnp.where(m, v, bs[u])
            ss[u] = jnp.where(m, i, ss[u])
        return tuple(bs), tuple(ss)

    init = (tuple(jnp.full((_L,), -jnp.inf, jnp.float32) for _ in range(U)),
            tuple(jnp.zeros((_L,), jnp.int32) for _ in range(U)))
    bs, ss = lax.fori_loop(0, _CHUNK // _L // U, body, init)

    lanes = lax.broadcasted_iota(jnp.int32, (_L,), 0)
    pairs = [(bs[u], (ss[u] << 7) + (u << 4) + lanes) for u in range(U)]
    while len(pairs) > 1:
        nxt = []
        for a in range(0, len(pairs), 2):
            (va, ia), (vb, ib) = pairs[a], pairs[a + 1]
            m = (va > vb) | ((va == vb) & (ia < ib))
            nxt.append((jnp.where(m, va, vb), jnp.where(m, ia, ib)))
        pairs = nxt
    return pairs[0]


def _sc_body(x1_hbm, x2_hbm, pval_hbm, pidx_hbm, rowval_hbm, rowidx_hbm,
             buf0, buf1, pvb, pib, rvb, rib, sem0, sem1):
    wid = lax.axis_index("s") * _NC + lax.axis_index("c")
    lanes = lax.broadcasted_iota(jnp.int32, (_L,), 0)
    bufs, sems = (buf0, buf1), (sem0, sem1)

    base1 = wid * _X1_PER_W
    descs = [
        (x1_hbm, 2 * wid, 0, 0),
        (x1_hbm, 2 * wid + 1, 0, _CHUNK),
        (x2_hbm, 2 * wid, 1, 0),
        (x2_hbm, 2 * wid + 1, 2, 0),
    ]
    nk = len(descs)
    handles = [None] * nk

    def issue(k):
        ref, row = descs[k][0], descs[k][1]
        handles[k] = pltpu.async_copy(ref.at[row], bufs[k % 2], sems[k % 2])

    issue(0)
    issue(1)
    seg = {}
    for k in range(nk):
        handles[k].wait()
        bc, ci = _scan_chunk(bufs[k % 2])
        if k + 2 < nk:
            issue(k + 2)
        absi = descs[k][3] + ci
        s = descs[k][2]
        if s in seg:
            b0, i0 = seg[s]
            m = bc > b0
            seg[s] = (jnp.where(m, bc, b0), jnp.where(m, absi, i0))
        else:
            seg[s] = (bc, absi)

    pvb[...] = seg[0][0]
    pib[...] = base1 + seg[0][1]
    for j in range(2):
        rvb[j] = seg[1 + j][0]
        rib[j] = seg[1 + j][1]
    pltpu.sync_copy(pvb, pval_hbm.at[wid])
    pltpu.sync_copy(pib, pidx_hbm.at[wid])
    pltpu.sync_copy(rvb, rowval_hbm.at[pl.ds(2 * wid, 2)])
    pltpu.sync_copy(rib, rowidx_hbm.at[pl.ds(2 * wid, 2)])


_sc_partial = functools.partial(
    pl.kernel,
    out_type=[
        jax.ShapeDtypeStruct((_NW, _L), jnp.float32),
        jax.ShapeDtypeStruct((_NW, _L), jnp.int32),
        jax.ShapeDtypeStruct((64, _L), jnp.float32),
        jax.ShapeDtypeStruct((64, _L), jnp.int32),
    ],
    mesh=plsc.VectorSubcoreMesh(core_axis_name="c", subcore_axis_name="s"),
    scratch_types=[
        pltpu.VMEM((_CHUNK,), jnp.float32),
        pltpu.VMEM((_CHUNK,), jnp.float32),
        pltpu.VMEM((_L,), jnp.float32),
        pltpu.VMEM((_L,), jnp.int32),
        pltpu.VMEM((2, _L), jnp.float32),
        pltpu.VMEM((2, _L), jnp.int32),
        pltpu.SemaphoreType.DMA,
        pltpu.SemaphoreType.DMA,
    ],
)(_sc_body)


def _merge_body(pv_ref, pi_ref, rv_ref, ri_ref, o_ref):
    mv = jnp.max(pv_ref[...])
    gidx = jnp.min(jnp.where(pv_ref[...] == mv, pi_ref[...], _BIG))
    rmax = jnp.max(rv_ref[...], axis=1, keepdims=True)
    rarg = jnp.min(jnp.where(rv_ref[...] == rmax, ri_ref[...], _BIG), axis=1)
    o_ref[...] = rarg + gidx


def kernel(x1, x2):
    pvals, pidx, rowval, rowidx = _sc_partial(x1, x2)
    return pl.pallas_call(
        _merge_body,
        out_shape=jax.ShapeDtypeStruct((64,), jnp.int32),
    )(pvals, pidx, rowval, rowidx)


# speedup vs baseline: 3.0876x; 1.0445x over previous
import functools

import jax
import jax.numpy as jnp
from jax import lax
from jax.experimental import pallas as pl
from jax.experimental.pallas import tpu as pltpu
from jax.experimental.pallas import tpu_sc as plsc

_L = 16
_NC, _NS = 2, 16
_NW = _NC * _NS
_X1_PER_W = 2 * 32768
_CHUNK = 32768
_UNROLL = 8
_BIG = 2**31 - 1


def _scan_chunk(buf):
    U = _UNROLL

    def body(i, carry):
        bs, ss = list(carry[0]), list(carry[1])
        for u in range(U):
            v = buf[pl.ds((i * U + u) * _L, _L)]
            m = v > bs[u]
            bs[u] = jnp.where(m, v, bs[u])
            ss[u] = jnp.where(m, i, ss[u])
        return tuple(bs), tuple(ss)

    init = (tuple(jnp.full((_L,), -jnp.inf, jnp.float32) for _ in range(U)),
            tuple(jnp.zeros((_L,), jnp.int32) for _ in range(U)))
    bs, ss = lax.fori_loop(0, _CHUNK // _L // U, body, init)

    lanes = lax.broadcasted_iota(jnp.int32, (_L,), 0)
    pairs = [(bs[u], (ss[u] << 7) + (u << 4) + lanes) for u in range(U)]
    while len(pairs) > 1:
        nxt = []
        for a in range(0, len(pairs), 2):
            (va, ia), (vb, ib) = pairs[a], pairs[a + 1]
            m = (va > vb) | ((va == vb) & (ia < ib))
            nxt.append((jnp.where(m, va, vb), jnp.where(m, ia, ib)))
        pairs = nxt
    return pairs[0]


def _sc_body(x1_hbm, pval_hbm, pidx_hbm, buf0, buf1, pvb, pib, sem0, sem1):
    wid = lax.axis_index("s") * _NC + lax.axis_index("c")
    bufs, sems = (buf0, buf1), (sem0, sem1)
    handles = [None, None]

    def issue(k):
        handles[k] = pltpu.async_copy(x1_hbm.at[2 * wid + k], bufs[k], sems[k])

    issue(0)
    issue(1)
    best = None
    for k in range(2):
        handles[k].wait()
        bc, ci = _scan_chunk(bufs[k])
        absi = k * _CHUNK + ci
        if best is None:
            best = (bc, absi)
        else:
            b0, i0 = best
            m = bc > b0
            best = (jnp.where(m, bc, b0), jnp.where(m, absi, i0))

    pvb[...] = best[0]
    pib[...] = wid * _X1_PER_W + best[1]
    pltpu.sync_copy(pvb, pval_hbm.at[wid])
    pltpu.sync_copy(pib, pidx_hbm.at[wid])


_sc_x1_partial = functools.partial(
    pl.kernel,
    out_type=[
        jax.ShapeDtypeStruct((_NW, _L), jnp.float32),
        jax.ShapeDtypeStruct((_NW, _L), jnp.int32),
    ],
    mesh=plsc.VectorSubcoreMesh(core_axis_name="c", subcore_axis_name="s"),
    scratch_types=[
        pltpu.VMEM((_CHUNK,), jnp.float32),
        pltpu.VMEM((_CHUNK,), jnp.float32),
        pltpu.VMEM((_L,), jnp.float32),
        pltpu.VMEM((_L,), jnp.int32),
        pltpu.SemaphoreType.DMA,
        pltpu.SemaphoreType.DMA,
    ],
)(_sc_body)


def _rows_body(x_ref, o_ref):
    x = x_ref[...]
    m = jnp.max(x, axis=1, keepdims=True)
    idx = lax.broadcasted_iota(jnp.int32, x.shape, 1)
    cand = jnp.where(x == m, idx, _BIG)
    o_ref[0, 0, :] = jnp.min(cand, axis=1)


def _merge_body(pv_ref, pi_ref, ra_ref, o_ref):
    mv = jnp.max(pv_ref[...])
    gidx = jnp.min(jnp.where(pv_ref[...] == mv, pi_ref[...], _BIG))
    o_ref[...] = ra_ref[...] + gidx


def kernel(x1, x2):
    pvals, pidx = _sc_x1_partial(x1)
    rowarg = pl.pallas_call(
        _rows_body,
        grid=(8,),
        in_specs=[pl.BlockSpec((8, 32768), lambda i: (i, 0))],
        out_specs=pl.BlockSpec((1, 1, 8), lambda i: (i, 0, 0)),
        out_shape=jax.ShapeDtypeStruct((8, 1, 8), jnp.int32),
    )(x2).reshape(64)
    return pl.pallas_call(
        _merge_body,
        out_shape=jax.ShapeDtypeStruct((64,), jnp.int32),
    )(pvals, pidx, rowarg)


# speedup vs baseline: 3.3052x; 1.0705x over previous
import functools

import jax
import jax.numpy as jnp
from jax import lax
from jax.experimental import pallas as pl
from jax.experimental.pallas import tpu as pltpu
from jax.experimental.pallas import tpu_sc as plsc

_L = 16
_NC, _NS = 2, 16
_NW = _NC * _NS
_ROW = 32768
_X1_PER_W = 2 * _ROW
_CHUNK = 16384
_NBUF = 3
_UNROLL = 8
_BIG = 2**31 - 1


def _scan_chunk(buf):
    U = _UNROLL

    def body(i, carry):
        bs, ss = list(carry[0]), list(carry[1])
        for u in range(U):
            v = buf[pl.ds((i * U + u) * _L, _L)]
            m = v > bs[u]
            bs[u] = jnp.where(m, v, bs[u])
            ss[u] = jnp.where(m, i, ss[u])
        return tuple(bs), tuple(ss)

    init = (tuple(jnp.full((_L,), -jnp.inf, jnp.float32) for _ in range(U)),
            tuple(jnp.zeros((_L,), jnp.int32) for _ in range(U)))
    bs, ss = lax.fori_loop(0, _CHUNK // _L // U, body, init)

    lanes = lax.broadcasted_iota(jnp.int32, (_L,), 0)
    pairs = [(bs[u], (ss[u] << 7) + (u << 4) + lanes) for u in range(U)]
    while len(pairs) > 1:
        nxt = []
        for a in range(0, len(pairs), 2):
            (va, ia), (vb, ib) = pairs[a], pairs[a + 1]
            m = (va > vb) | ((va == vb) & (ia < ib))
            nxt.append((jnp.where(m, va, vb), jnp.where(m, ia, ib)))
        pairs = nxt
    return pairs[0]


def _sc_body(x1_hbm, pval_hbm, pidx_hbm,
             buf0, buf1, buf2, pvb, pib, sem0, sem1, sem2):
    wid = lax.axis_index("s") * _NC + lax.axis_index("c")
    bufs, sems = (buf0, buf1, buf2), (sem0, sem1, sem2)
    nch = _X1_PER_W // _CHUNK
    per_row = _ROW // _CHUNK
    handles = [None] * nch

    def issue(k):
        row = 2 * wid + k // per_row
        col = (k % per_row) * _CHUNK
        handles[k] = pltpu.async_copy(x1_hbm.at[row, pl.ds(col, _CHUNK)],
                                      bufs[k % _NBUF], sems[k % _NBUF])

    for k in range(_NBUF):
        issue(k)
    best = None
    for k in range(nch):
        handles[k].wait()
        bc, ci = _scan_chunk(bufs[k % _NBUF])
        if k + _NBUF < nch:
            issue(k + _NBUF)
        absi = k * _CHUNK + ci
        if best is None:
            best = (bc, absi)
        else:
            b0, i0 = best
            m = bc > b0
            best = (jnp.where(m, bc, b0), jnp.where(m, absi, i0))

    pvb[...] = best[0]
    pib[...] = wid * _X1_PER_W + best[1]
    cv = pltpu.async_copy(pvb, pval_hbm.at[wid], sem0)
    ci_ = pltpu.async_copy(pib, pidx_hbm.at[wid], sem1)
    cv.wait()
    ci_.wait()


_sc_x1_partial = functools.partial(
    pl.kernel,
    out_type=[
        jax.ShapeDtypeStruct((_NW, _L), jnp.float32),
        jax.ShapeDtypeStruct((_NW, _L), jnp.int32),
    ],
    mesh=plsc.VectorSubcoreMesh(core_axis_name="c", subcore_axis_name="s"),
    scratch_types=[
        pltpu.VMEM((_CHUNK,), jnp.float32),
        pltpu.VMEM((_CHUNK,), jnp.float32),
        pltpu.VMEM((_CHUNK,), jnp.float32),
        pltpu.VMEM((_L,), jnp.float32),
        pltpu.VMEM((_L,), jnp.int32),
        pltpu.SemaphoreType.DMA,
        pltpu.SemaphoreType.DMA,
        pltpu.SemaphoreType.DMA,
    ],
)(_sc_body)


def _rows_body(x_ref, o_ref):
    i = pl.program_id(0)
    x = x_ref[...]
    m = jnp.max(x, axis=1, keepdims=True)
    idx = lax.broadcasted_iota(jnp.int32, x.shape, 1)
    cand = jnp.where(x == m, idx, _BIG)
    rarg = jnp.min(cand, axis=1)
    for j in range(8):
        @pl.when(i == j)
        def _():
            o_ref[j * 8:(j + 1) * 8] = rarg


def _merge_body(pv_ref, pi_ref, ra_ref, o_ref):
    mv = jnp.max(pv_ref[...])
    gidx = jnp.min(jnp.where(pv_ref[...] == mv, pi_ref[...], _BIG))
    o_ref[...] = ra_ref[...] + gidx


def kernel(x1, x2):
    pvals, pidx = _sc_x1_partial(x1)
    rowarg = pl.pallas_call(
        _rows_body,
        grid=(8,),
        in_specs=[pl.BlockSpec((8, _ROW), lambda i: (i, 0))],
        out_specs=pl.BlockSpec((64,), lambda i: (0,)),
        out_shape=jax.ShapeDtypeStruct((64,), jnp.int32),
    )(x2)
    return pl.pallas_call(
        _merge_body,
        out_shape=jax.ShapeDtypeStruct((64,), jnp.int32),
    )(pvals, pidx, rowarg)
